# Initial kernel scaffold; baseline (speedup 1.0000x reference)
import jax
import jax.numpy as jnp
from jax.experimental import pallas as pl


def kernel(token_ids, emb):
    raise NotImplementedError("write your pallas kernel here")


# speedup vs baseline: 1.1125x; 1.1125x over previous
import functools

import jax
import jax.numpy as jnp
from jax import lax
from jax.experimental import pallas as pl
from jax.experimental.pallas import tpu as pltpu
from jax.experimental.pallas import tpu_sc as plsc

_B = 16384 * 50
_D = 32
_C = 512


def _make_lookup():
    info = plsc.get_sparse_core_info()
    nc, ns = info.num_cores, info.num_subcores
    nw = nc * ns
    b_per_w = _B // nw
    n_chunks = b_per_w // _C
    n_pairs = n_chunks // 2
    mesh = plsc.VectorSubcoreMesh(core_axis_name="c", subcore_axis_name="s")

    @functools.partial(
        pl.kernel,
        out_type=jax.ShapeDtypeStruct((_B, _D), jnp.float32),
        mesh=mesh,
        compiler_params=pltpu.CompilerParams(use_tc_tiling_on_sc=False),
        scratch_types=[
            pltpu.VMEM((b_per_w,), jnp.int32),
            pltpu.VMEM((_C, _D), jnp.float32),
            pltpu.VMEM((_C, _D), jnp.float32),
            pltpu.SemaphoreType.DMA,
            pltpu.SemaphoreType.DMA,
        ],
    )
    def lookup(ids_hbm, emb_hbm, out_hbm, idx_v, buf0, buf1, sem0, sem1):
        wid = lax.axis_index("s") * nc + lax.axis_index("c")
        base = wid * b_per_w
        pltpu.sync_copy(ids_hbm.at[pl.ds(base, b_per_w)], idx_v)

        def gather(i, buf, sem):
            pltpu.async_copy(emb_hbm.at[idx_v.at[pl.ds(i * _C, _C)]], buf, sem)

        def wait_gather(i, buf, sem):
            pltpu.make_async_copy(
                emb_hbm.at[idx_v.at[pl.ds(i * _C, _C)]], buf, sem
            ).wait()

        def store(i, buf):
            pltpu.sync_copy(buf, out_hbm.at[pl.ds(base + i * _C, _C)])

        gather(0, buf0, sem0)

        def body(j, carry):
            i0 = 2 * j
            gather(i0 + 1, buf1, sem1)
            wait_gather(i0, buf0, sem0)
            store(i0, buf0)

            @pl.when(j < n_pairs - 1)
            def _():
                gather(i0 + 2, buf0, sem0)

            wait_gather(i0 + 1, buf1, sem1)
            store(i0 + 1, buf1)
            return carry

        lax.fori_loop(0, n_pairs, body, 0)

    return lookup


_lookup = _make_lookup()


@jax.jit
def kernel(token_ids, emb):
    ids = token_ids.reshape(-1).astype(jnp.int32)
    out = _lookup(ids, emb)
    return out.reshape(token_ids.shape + (_D,))


# speedup vs baseline: 1.4335x; 1.2886x over previous
import functools

import jax
import jax.numpy as jnp
from jax import lax
from jax.experimental import pallas as pl
from jax.experimental.pallas import tpu as pltpu
from jax.experimental.pallas import tpu_sc as plsc

_B = 16384
_T = 50
_D = 32
_CH = 256
_V = 1000000


def _make_lookup():
    info = plsc.get_sparse_core_info()
    nc, ns = info.num_cores, info.num_subcores
    nw = nc * ns
    b_per_w = _B // nw
    n_ids = _T * b_per_w
    n_chunks = n_ids // _CH
    n_pairs = n_chunks // 2
    mesh = plsc.VectorSubcoreMesh(core_axis_name="c", subcore_axis_name="s")

    @functools.partial(
        pl.kernel,
        out_type=jax.ShapeDtypeStruct((_T * _D, _B), jnp.float32),
        mesh=mesh,
        compiler_params=pltpu.CompilerParams(
            use_tc_tiling_on_sc=False, needs_layout_passes=False
        ),
        scratch_types=[
            pltpu.VMEM((n_ids,), jnp.int32),
            pltpu.VMEM((2 * _CH,), jnp.int32),
            pltpu.VMEM((2 * _CH,), jnp.int32),
            pltpu.VMEM((2 * _CH, 16), jnp.float32),
            pltpu.VMEM((2 * _CH, 16), jnp.float32),
            pltpu.VMEM((_D, _CH), jnp.float32),
            pltpu.VMEM((_D, _CH), jnp.float32),
            pltpu.SemaphoreType.DMA,
            pltpu.SemaphoreType.DMA,
            pltpu.SemaphoreType.DMA,
            pltpu.SemaphoreType.DMA,
            pltpu.SemaphoreType.DMA,
        ],
    )
    def lookup(ids_hbm, packs_hbm, out_hbm,
               ids_v, ip0, ip1, buf0, buf1, ob0, ob1,
               sem_i, sem_g0, sem_g1, sem_s0, sem_s1):
        wid = lax.axis_index("s") * nc + lax.axis_index("c")
        col0 = wid * b_per_w

        def stage(t, carry):
            pltpu.async_copy(
                ids_hbm.at[pl.ds(t * _B + col0, b_per_w)],
                ids_v.at[pl.ds(t * b_per_w, b_per_w)],
                sem_i,
            )
            return carry

        lax.fori_loop(0, _T, stage, 0)
        pltpu.make_async_copy(ids_hbm.at[pl.ds(0, n_ids)], ids_v, sem_i).wait()

        iota16 = lax.iota(jnp.int32, 16)
        cols = [iota16 * 0 + dm for dm in range(16)]
        rows_e = [2 * (iota16 + g * 16) for g in range(_CH // 16)]
        scat_e = [2 * iota16 + 32 * g for g in range(_CH // 16)]

        def build_ip(c, ip):
            for g in range(_CH // 16):
                v = ids_v[pl.ds(c * _CH + g * 16, 16)]
                v2 = v + v
                plsc.store_scatter(ip, [scat_e[g]], v2)
                plsc.store_scatter(ip, [scat_e[g] + 1], v2 + 1)

        def gather(ip, buf, sem):
            pltpu.async_copy(packs_hbm.at[ip], buf, sem)

        def wait_g(ip, buf, sem):
            pltpu.make_async_copy(packs_hbm.at[ip], buf, sem).wait()

        def transpose(buf, ob):
            for g in range(_CH // 16):
                re = rows_e[g]
                ro = re + 1
                for d in range(_D):
                    rows = re if d < 16 else ro
                    x = plsc.load_gather(buf, [rows, cols[d % 16]])
                    ob[d, pl.ds(g * 16, 16)] = x

        def out_slice(c):
            t = c // 2
            b0 = col0 + (c % 2) * _CH
            return out_hbm.at[pl.ds(_D * t, _D), pl.ds(b0, _CH)]

        def store(c, ob, sem):
            pltpu.async_copy(ob, out_slice(c), sem)

        def wait_s(c, ob, sem):
            pltpu.make_async_copy(ob, out_slice(c), sem).wait()

        build_ip(0, ip0)
        gather(ip0, buf0, sem_g0)

        def body(j, carry):
            c0 = 2 * j
            build_ip(c0 + 1, ip1)
            gather(ip1, buf1, sem_g1)
            wait_g(ip0, buf0, sem_g0)

            @pl.when(j > 0)
            def _():
                wait_s(c0 - 2, ob0, sem_s0)

            transpose(buf0, ob0)
            store(c0, ob0, sem_s0)

            @pl.when(j < n_pairs - 1)
            def _():
                build_ip(c0 + 2, ip0)
                gather(ip0, buf0, sem_g0)

            wait_g(ip1, buf1, sem_g1)

            @pl.when(j > 0)
            def _():
                wait_s(c0 - 1, ob1, sem_s1)

            transpose(buf1, ob1)
            store(c0 + 1, ob1, sem_s1)
            return carry

        lax.fori_loop(0, n_pairs, body, 0)
        wait_s(2 * n_pairs - 2, ob0, sem_s0)
        wait_s(2 * n_pairs - 1, ob1, sem_s1)

    return lookup


_lookup = _make_lookup()


@jax.jit
def kernel(token_ids, emb):
    ids_t_major = token_ids.T.reshape(-1).astype(jnp.int32)
    packs = emb.reshape(2 * _V, 16)
    out2 = _lookup(ids_t_major, packs)
    return jnp.transpose(out2.reshape(_T, _D, _B), (2, 0, 1))
